# Initial kernel scaffold; baseline (speedup 1.0000x reference)
import jax
import jax.numpy as jnp
from jax.experimental import pallas as pl


def kernel(x, conv_w1, conv_b1, conv_g1, conv_be1, conv_w2, conv_b2, conv_g2, conv_be2, conv12_w1, conv12_b1, conv12_g1, conv12_be1, conv12_w2, conv12_b2, conv12_g2, conv12_be2, conv22_w1, conv22_b1, conv22_g1, conv22_be1, conv22_w2, conv22_b2, conv22_g2, conv22_be2, conv_am11_w, mlp_w1, mlp_b1, mlp_w2, mlp_b2, conv_am21_w, gat1_w, gat1_al, gat1_ar, gat1_b, gat2_w, gat2_al, gat2_ar, gat2_b, gat3_w, gat3_al, gat3_ar, gat3_b, conv_am_end_w):
    raise NotImplementedError("write your pallas kernel here")


# speedup vs baseline: 1.7939x; 1.7939x over previous
import jax
import jax.numpy as jnp
from jax import lax
from jax.experimental import pallas as pl

_C = 64
_HH = 36
_B = 2
_HW = _HH * _HH
_N = _HW
_F32 = jnp.float32
_BF16 = jnp.bfloat16


def _lrelu(v, s):
    return jnp.where(v >= 0, v, s * v)


def _conv3x3(h, w9_ref, b_ref):
    pos = lax.broadcasted_iota(jnp.int32, (1, _HW, 1), 1)
    r = pos // _HH
    c = pos % _HH
    hb = h.astype(_BF16)
    acc = None
    for dy in (-1, 0, 1):
        for dx in (-1, 0, 1):
            k = (dy + 1) * 3 + (dx + 1)
            sh = dy * _HH + dx
            shifted = jnp.roll(hb, -sh, axis=1) if sh != 0 else hb
            valid = ((r + dy >= 0) & (r + dy < _HH)
                     & (c + dx >= 0) & (c + dx < _HH))
            masked = jnp.where(valid, shifted, _BF16(0))
            term = lax.dot_general(
                masked.reshape(_B * _HW, _C), w9_ref[k],
                (((1,), (0,)), ((), ())), preferred_element_type=_F32)
            acc = term if acc is None else acc + term
    return acc.reshape(_B, _HW, _C) + b_ref[...]


def _bn(h, g_ref, be_ref):
    m = jnp.mean(h, axis=(0, 1), keepdims=True)
    v = jnp.mean((h - m) * (h - m), axis=(0, 1), keepdims=True)
    return (h - m) * lax.rsqrt(v + 1e-5) * g_ref[...] + be_ref[...]


def _dual(h, wa, ba, ga, bea, wb, bb, gb, beb):
    h = jax.nn.relu(_bn(_conv3x3(h, wa, ba), ga, bea))
    h = jax.nn.relu(_bn(_conv3x3(h, wb, bb), gb, beb))
    return h


def _gat_layer(h, acol, arow, w_ref, alb_ref, arb_ref, b_ref,
               heads, odim, act, mean_heads):
    feat = lax.dot_general(h, w_ref[...], (((1,), (1,)), ((), ())),
                           preferred_element_type=_F32)
    elr = lax.dot_general(alb_ref[...], feat, (((1,), (1,)), ((), ())),
                          preferred_element_type=_F32)
    elc = lax.dot_general(feat, alb_ref[...], (((1,), (1,)), ((), ())),
                          preferred_element_type=_F32)
    erc = lax.dot_general(feat, arb_ref[...], (((1,), (1,)), ((), ())),
                          preferred_element_type=_F32)

    flag = jnp.max(jnp.abs(arow)) > 0.5
    thr = jnp.where(flag, _F32(0.25), _F32(jnp.inf))
    adj = (acol * arow) > thr

    outs = []
    acc = None
    for hh in range(heads):
        lo = hh * odim
        feat_h = feat[:, lo:lo + odim]
        el = elr[hh:hh + 1, :]
        er = erc[:, hh:hh + 1]
        eld = elc[:, hh:hh + 1]
        m = _lrelu(jnp.max(el) + er, 0.2)
        exd = jnp.exp(_lrelu(eld + er, 0.2) - m)
        e = _lrelu(er + el, 0.2)
        ex = jnp.where(adj, jnp.exp(e - m), 0.0)
        denom = jnp.sum(ex, axis=1, keepdims=True) + exd
        rst = lax.dot_general(ex, feat_h, (((1,), (0,)), ((), ())),
                              preferred_element_type=_F32)
        rst = (rst + exd * feat_h) / denom + b_ref[:, lo:lo + odim]
        if act:
            rst = jnp.where(rst > 0, rst, jnp.exp(rst) - 1.0)
        if mean_heads:
            acc = rst if acc is None else acc + rst
        else:
            outs.append(rst)
    if mean_heads:
        return acc * (1.0 / heads)
    return jnp.concatenate(outs, axis=1)


def _front_body(x_ref,
                cw1, cb1, cg1, cbe1, cw2, cb2, cg2, cbe2,
                dw1, db1, dg1, dbe1, dw2, db2, dg2, dbe2,
                am11, mw1, mb1, mw2, mb2, am21,
                x1_out, x2_out):
    X = x_ref[...]
    x = _dual(X, cw1, cb1, cg1, cbe1, cw2, cb2, cg2, cbe2)
    xb = x.astype(_BF16).astype(_F32)

    s1 = jnp.sum(xb * am11[...].astype(_BF16).astype(_F32),
                 axis=2, keepdims=True)
    x1 = x * _lrelu(s1, 0.01)
    s = (jnp.max(x1, axis=1, keepdims=True)
         + jnp.mean(x1, axis=1, keepdims=True)).reshape(_B, _C)
    t = jax.nn.relu(lax.dot_general(s.astype(_BF16), mw1[...],
                                    (((1,), (1,)), ((), ())),
                                    preferred_element_type=_F32) + mb1[...])
    sg = jax.nn.sigmoid(lax.dot_general(t.astype(_BF16), mw2[...],
                                        (((1,), (1,)), ((), ())),
                                        preferred_element_type=_F32)
                        + mb2[...])
    x1 = x1 * sg.reshape(_B, 1, _C)
    x1 = _dual(x1, dw1, db1, dg1, dbe1, dw2, db2, dg2, dbe2)

    s2 = jnp.sum(xb * am21[...].astype(_BF16).astype(_F32),
                 axis=2, keepdims=True)
    x2 = x * _lrelu(s2, 0.01)
    x1_out[...] = x1
    x2_out[...] = x2


def _gat_body(f_ref,
              w1_ref, alb1, arb1, b1_ref,
              w2_ref, alb2, arb2, b2_ref,
              w3_ref, alb3, arb3, b3_ref,
              u_out):
    f = f_ref[0]
    ones = jnp.full((1, _C), 1.0 / _C, _F32)
    acol = jnp.mean(f, axis=1, keepdims=True)
    arow = lax.dot_general(ones, f, (((1,), (1,)), ((), ())),
                           preferred_element_type=_F32)
    h = _gat_layer(f, acol, arow, w1_ref, alb1, arb1, b1_ref,
                   3, 2 * _C, True, False)
    h = _gat_layer(h, acol, arow, w2_ref, alb2, arb2, b2_ref,
                   5, 2 * _C, True, False)
    h = _gat_layer(h, acol, arow, w3_ref, alb3, arb3, b3_ref,
                   3, _C, False, True)
    u_out[0] = h


def _back_body(u_ref, x1_ref,
               ew1, eb1, eg1, ebe1, ew2, eb2, eg2, ebe2,
               wa_ref, wb_ref, out_ref):
    u = u_ref[...]
    x1 = x1_ref[...]
    u = _dual(u, ew1, eb1, eg1, ebe1, ew2, eb2, eg2, ebe2)
    o = (lax.dot_general(x1.astype(_BF16).reshape(_B * _HW, _C),
                         wa_ref[...], (((1,), (0,)), ((), ())),
                         preferred_element_type=_F32)
         + lax.dot_general(u.astype(_BF16).reshape(_B * _HW, _C),
                           wb_ref[...], (((1,), (0,)), ((), ())),
                           preferred_element_type=_F32))
    out_ref[...] = _lrelu(o, 0.01).reshape(_B, _HW, _C)


def _w9(w):
    return w.transpose(2, 3, 1, 0).reshape(9, _C, _C).astype(_BF16)


def _vec(v):
    return v.reshape(1, 1, -1)


def _blockdiag(a, odim):
    heads = a.shape[0]
    out = jnp.zeros((heads, heads * odim), a.dtype)
    for hh in range(heads):
        out = out.at[hh, hh * odim:(hh + 1) * odim].set(a[hh])
    return out


def kernel(x, conv_w1, conv_b1, conv_g1, conv_be1, conv_w2, conv_b2, conv_g2,
           conv_be2, conv12_w1, conv12_b1, conv12_g1, conv12_be1, conv12_w2,
           conv12_b2, conv12_g2, conv12_be2, conv22_w1, conv22_b1, conv22_g1,
           conv22_be1, conv22_w2, conv22_b2, conv22_g2, conv22_be2,
           conv_am11_w, mlp_w1, mlp_b1, mlp_w2, mlp_b2, conv_am21_w,
           gat1_w, gat1_al, gat1_ar, gat1_b,
           gat2_w, gat2_al, gat2_ar, gat2_b,
           gat3_w, gat3_al, gat3_ar, gat3_b,
           conv_am_end_w):
    X = x.transpose(0, 2, 3, 1).reshape(_B, _HW, _C)
    we = conv_am_end_w.reshape(_C, 2 * _C).astype(_BF16)
    params = (
        _w9(conv_w1), _vec(conv_b1), _vec(conv_g1), _vec(conv_be1),
        _w9(conv_w2), _vec(conv_b2), _vec(conv_g2), _vec(conv_be2),
        _w9(conv12_w1), _vec(conv12_b1), _vec(conv12_g1), _vec(conv12_be1),
        _w9(conv12_w2), _vec(conv12_b2), _vec(conv12_g2), _vec(conv12_be2),
        _vec(conv_am11_w.reshape(-1)),
        mlp_w1.reshape(_C // 4, _C).astype(_BF16), mlp_b1.reshape(1, -1),
        mlp_w2.reshape(_C, _C // 4).astype(_BF16), mlp_b2.reshape(1, -1),
        _vec(conv_am21_w.reshape(-1)),
    )
    x1, x2 = pl.pallas_call(
        _front_body,
        out_shape=(jax.ShapeDtypeStruct((_B, _HW, _C), _F32),
                   jax.ShapeDtypeStruct((_B, _HW, _C), _F32)),
    )(X, *params)

    gat_params = (
        gat1_w, _blockdiag(gat1_al, 2 * _C), _blockdiag(gat1_ar, 2 * _C),
        gat1_b.reshape(1, -1),
        gat2_w, _blockdiag(gat2_al, 2 * _C), _blockdiag(gat2_ar, 2 * _C),
        gat2_b.reshape(1, -1),
        gat3_w, _blockdiag(gat3_al, _C), _blockdiag(gat3_ar, _C),
        gat3_b.reshape(1, -1),
    )
    gspecs = [pl.BlockSpec((1, _HW, _C), lambda i: (i, 0, 0))]
    gspecs += [pl.BlockSpec(p.shape, lambda i: tuple(0 for _ in p.shape))
               for p in gat_params]
    u = pl.pallas_call(
        _gat_body,
        grid=(_B,),
        in_specs=gspecs,
        out_specs=pl.BlockSpec((1, _HW, _C), lambda i: (i, 0, 0)),
        out_shape=jax.ShapeDtypeStruct((_B, _HW, _C), _F32),
    )(x2, *gat_params)

    back_params = (
        _w9(conv22_w1), _vec(conv22_b1), _vec(conv22_g1), _vec(conv22_be1),
        _w9(conv22_w2), _vec(conv22_b2), _vec(conv22_g2), _vec(conv22_be2),
        we[:, :_C].T, we[:, _C:].T,
    )
    out = pl.pallas_call(
        _back_body,
        out_shape=jax.ShapeDtypeStruct((_B, _HW, _C), _F32),
    )(u, x1, *back_params)
    return out.reshape(_B, _HH, _HH, _C).transpose(0, 3, 1, 2)


# speedup vs baseline: 1.9279x; 1.0747x over previous
import functools

import jax
import jax.numpy as jnp
from jax import lax
from jax.experimental import pallas as pl
from jax.experimental.pallas import tpu as pltpu

_C = 64
_HH = 36
_B = 2
_HW = _HH * _HH
_NP = 1408
_T = 1408
_NT = _NP // _T
_F32 = jnp.float32


def _lrelu(v, s):
    return jnp.where(v >= 0, v, s * v)


def _conv3x3(h, w9_ref, b_ref):
    pos = lax.broadcasted_iota(jnp.int32, (1, _HW, 1), 1)
    r = pos // _HH
    c = pos % _HH
    hb = h.astype(jnp.bfloat16)
    acc = None
    for dy in (-1, 0, 1):
        for dx in (-1, 0, 1):
            k = (dy + 1) * 3 + (dx + 1)
            sh = dy * _HH + dx
            shifted = jnp.roll(hb, -sh, axis=1) if sh != 0 else hb
            valid = ((r + dy >= 0) & (r + dy < _HH)
                     & (c + dx >= 0) & (c + dx < _HH))
            masked = jnp.where(valid, shifted, jnp.bfloat16(0))
            term = lax.dot_general(
                masked.reshape(_B * _HW, _C), w9_ref[k],
                (((1,), (0,)), ((), ())), preferred_element_type=_F32)
            acc = term if acc is None else acc + term
    return acc.reshape(_B, _HW, _C) + b_ref[...]


def _bn(h, g_ref, be_ref):
    m = jnp.mean(h, axis=(0, 1), keepdims=True)
    v = jnp.mean((h - m) * (h - m), axis=(0, 1), keepdims=True)
    return (h - m) * lax.rsqrt(v + 1e-5) * g_ref[...] + be_ref[...]


def _dual(h, wa, ba, ga, bea, wb, bb, gb, beb):
    h = jax.nn.relu(_bn(_conv3x3(h, wa, ba), ga, bea))
    h = jax.nn.relu(_bn(_conv3x3(h, wb, bb), gb, beb))
    return h


def _frontend_body(x_ref,
                   cw1, cb1, cg1, cbe1, cw2, cb2, cg2, cbe2,
                   dw1, db1, dg1, dbe1, dw2, db2, dg2, dbe2,
                   am11, mw1, mb1, mw2, mb2, am21,
                   x1_out, f_out, acol_out, arow_out):
    X = x_ref[...]
    x = _dual(X, cw1, cb1, cg1, cbe1, cw2, cb2, cg2, cbe2)
    xb = x.astype(jnp.bfloat16).astype(_F32)

    s1 = jnp.sum(xb * am11[...].astype(jnp.bfloat16).astype(_F32),
                 axis=2, keepdims=True)
    x1 = x * _lrelu(s1, 0.01)
    s = (jnp.max(x1, axis=1, keepdims=True)
         + jnp.mean(x1, axis=1, keepdims=True)).reshape(_B, _C)
    t = jax.nn.relu(lax.dot_general(s.astype(jnp.bfloat16), mw1[...],
                                    (((1,), (1,)), ((), ())),
                                    preferred_element_type=_F32) + mb1[...])
    sg = jax.nn.sigmoid(lax.dot_general(t.astype(jnp.bfloat16), mw2[...],
                                        (((1,), (1,)), ((), ())),
                                        preferred_element_type=_F32) + mb2[...])
    x1 = x1 * sg.reshape(_B, 1, _C)
    x1 = _dual(x1, dw1, db1, dg1, dbe1, dw2, db2, dg2, dbe2)
    x1_out[...] = x1

    s2 = jnp.sum(xb * am21[...].astype(jnp.bfloat16).astype(_F32),
                 axis=2, keepdims=True)
    x2 = x * _lrelu(s2, 0.01)
    zf = jnp.zeros((_B, _NP - _HW, _C), _F32)
    f_out[...] = jnp.concatenate([x2, zf], axis=1)

    acol = jnp.mean(x2, axis=2, keepdims=True)
    za = jnp.zeros((_B, _NP - _HW, 1), _F32)
    acol_out[...] = jnp.concatenate([acol, za], axis=1)
    ones = jnp.full((1, _C), 1.0 / _C, _F32)
    rows = [lax.dot_general(ones, x2[i], (((1,), (1,)), ((), ())),
                            preferred_element_type=_F32,
                             precision=lax.Precision.HIGHEST)
            for i in range(_B)]
    arow = jnp.concatenate(rows, axis=0)
    zr = jnp.zeros((_B, _NP - _HW), _F32)
    arow_out[...] = jnp.concatenate([arow, zr], axis=1).reshape(_B, 1, _NP)


def _frontend(X, p9):
    outs = (
        jax.ShapeDtypeStruct((_B, _HW, _C), _F32),
        jax.ShapeDtypeStruct((_B, _NP, _C), _F32),
        jax.ShapeDtypeStruct((_B, _NP, 1), _F32),
        jax.ShapeDtypeStruct((_B, 1, _NP), _F32),
    )
    return pl.pallas_call(_frontend_body, out_shape=outs)(X, *p9)


def _gat_body(h_ref, acol_ref, arow_ref, w_ref, alb_ref, arb_ref, b_ref,
              out_ref, feat_ref, elr_ref, elc_ref, erc_ref,
              bex_ref, dex_ref, emax_ref,
              *, heads, odim, act, mean_heads):
    j = pl.program_id(1)

    @pl.when(j == 0)
    def _():
        feat = lax.dot_general(h_ref[0], w_ref[...], (((1,), (1,)), ((), ())),
                               preferred_element_type=_F32)
        feat_ref[...] = feat
        elr = lax.dot_general(alb_ref[...], feat, (((1,), (1,)), ((), ())),
                              preferred_element_type=_F32)
        elr_ref[...] = elr
        elc_ref[...] = lax.dot_general(feat, alb_ref[...],
                                       (((1,), (1,)), ((), ())),
                                       preferred_element_type=_F32)
        erc_ref[...] = lax.dot_general(feat, arb_ref[...],
                                       (((1,), (1,)), ((), ())),
                                       preferred_element_type=_F32)
        elmax = jnp.max(elr, axis=1, keepdims=True)
        bex_ref[...] = jnp.exp(elr - elmax)
        dex_ref[...] = jnp.exp(0.2 * (elr - elmax))
        emax_ref[...] = elmax

    d0 = j * _T
    arow = arow_ref[0]
    acol_t = acol_ref[0]
    flag = jnp.max(jnp.abs(arow)) > 0.5
    thr = jnp.where(flag, _F32(0.25), _F32(jnp.inf))
    prod = acol_t * arow
    adj = prod > thr

    acc = None
    for hh in range(heads):
        lo = hh * odim
        feat_h = feat_ref[:, lo:lo + odim]
        feat_t = feat_ref[pl.ds(d0, _T), lo:lo + odim]
        el = elr_ref[hh:hh + 1, :]
        er = erc_ref[pl.ds(d0, _T), hh:hh + 1]
        eld = elc_ref[pl.ds(d0, _T), hh:hh + 1]
        spe = emax_ref[hh:hh + 1, :] + er
        m = _lrelu(spe, 0.2)
        acoef = jnp.exp(spe - m)
        ccoef = jnp.exp(0.2 * spe - m)
        exd = jnp.exp(_lrelu(eld + er, 0.2) - m)
        bex = bex_ref[hh:hh + 1, :]
        dex = dex_ref[hh:hh + 1, :]
        pos = (er + el) >= 0
        ex0 = jnp.where(pos, acoef * bex, ccoef * dex)
        ex = jnp.where(adj, ex0, 0.0)
        denom = jnp.sum(ex, axis=1, keepdims=True) + exd
        rst = lax.dot_general(ex, feat_h, (((1,), (0,)), ((), ())),
                              preferred_element_type=_F32)
        rst = (rst + exd * feat_t) / denom + b_ref[:, lo:lo + odim]
        if act:
            rst = jnp.where(rst > 0, rst, jnp.exp(rst) - 1.0)
        if mean_heads:
            acc = rst if acc is None else acc + rst
        else:
            out_ref[0, :, lo:lo + odim] = rst
    if mean_heads:
        out_ref[0, :, :] = acc * (1.0 / heads)


def _blockdiag(a, odim):
    heads = a.shape[0]
    out = jnp.zeros((heads, heads * odim), a.dtype)
    for hh in range(heads):
        out = out.at[hh, hh * odim:(hh + 1) * odim].set(a[hh])
    return out


def _gat_layer(h, acol, arow, W, al, ar, b, heads, odim, width, act,
               mean_heads):
    body = functools.partial(_gat_body, heads=heads, odim=odim, act=act,
                             mean_heads=mean_heads)
    alb = _blockdiag(al, odim)
    arb = _blockdiag(ar, odim)
    return pl.pallas_call(
        body,
        grid=(_B, _NT),
        in_specs=[
            pl.BlockSpec((1, _NP, h.shape[2]), lambda i, j: (i, 0, 0)),
            pl.BlockSpec((1, _T, 1), lambda i, j: (i, j, 0)),
            pl.BlockSpec((1, 1, _NP), lambda i, j: (i, 0, 0)),
            pl.BlockSpec(W.shape, lambda i, j: (0, 0)),
            pl.BlockSpec(alb.shape, lambda i, j: (0, 0)),
            pl.BlockSpec(arb.shape, lambda i, j: (0, 0)),
            pl.BlockSpec(b.shape, lambda i, j: (0, 0)),
        ],
        out_specs=pl.BlockSpec((1, _T, width), lambda i, j: (i, j, 0)),
        out_shape=jax.ShapeDtypeStruct((_B, _NP, width), _F32),
        scratch_shapes=[pltpu.VMEM((_NP, W.shape[0]), _F32),
                        pltpu.VMEM((heads, _NP), _F32),
                        pltpu.VMEM((_NP, heads), _F32),
                        pltpu.VMEM((_NP, heads), _F32),
                        pltpu.VMEM((heads, _NP), _F32),
                        pltpu.VMEM((heads, _NP), _F32),
                        pltpu.VMEM((heads, 1), _F32)],
        compiler_params=pltpu.CompilerParams(
            dimension_semantics=("arbitrary", "arbitrary")),
    )(h, acol, arow, W, alb, arb, b)


def _backend_body(u_ref, x1_ref,
                  ew1, eb1, eg1, ebe1, ew2, eb2, eg2, ebe2,
                  wa_ref, wb_ref, out_ref):
    u = u_ref[...][:, :_HW, :]
    u = _dual(u, ew1, eb1, eg1, ebe1, ew2, eb2, eg2, ebe2)
    x1 = x1_ref[...]
    o = (lax.dot_general(x1.astype(jnp.bfloat16).reshape(_B * _HW, _C),
                         wa_ref[...], (((1,), (0,)), ((), ())),
                         preferred_element_type=_F32)
         + lax.dot_general(u.astype(jnp.bfloat16).reshape(_B * _HW, _C),
                           wb_ref[...], (((1,), (0,)), ((), ())),
                           preferred_element_type=_F32))
    out_ref[...] = _lrelu(o, 0.01).reshape(_B, _HW, _C)


def _backend(u, x1, p9, wa, wb):
    return pl.pallas_call(
        _backend_body,
        out_shape=jax.ShapeDtypeStruct((_B, _HW, _C), _F32),
    )(u, x1, *p9, wa, wb)


def _w9(w):
    return w.transpose(2, 3, 1, 0).reshape(9, _C, _C).astype(jnp.bfloat16)


def _vec(v):
    return v.reshape(1, 1, -1)


def kernel(x, conv_w1, conv_b1, conv_g1, conv_be1, conv_w2, conv_b2, conv_g2,
           conv_be2, conv12_w1, conv12_b1, conv12_g1, conv12_be1, conv12_w2,
           conv12_b2, conv12_g2, conv12_be2, conv22_w1, conv22_b1, conv22_g1,
           conv22_be1, conv22_w2, conv22_b2, conv22_g2, conv22_be2,
           conv_am11_w, mlp_w1, mlp_b1, mlp_w2, mlp_b2, conv_am21_w,
           gat1_w, gat1_al, gat1_ar, gat1_b,
           gat2_w, gat2_al, gat2_ar, gat2_b,
           gat3_w, gat3_al, gat3_ar, gat3_b,
           conv_am_end_w):
    X = x.transpose(0, 2, 3, 1).reshape(_B, _HW, _C)
    front_params = (
        _w9(conv_w1), _vec(conv_b1), _vec(conv_g1), _vec(conv_be1),
        _w9(conv_w2), _vec(conv_b2), _vec(conv_g2), _vec(conv_be2),
        _w9(conv12_w1), _vec(conv12_b1), _vec(conv12_g1), _vec(conv12_be1),
        _w9(conv12_w2), _vec(conv12_b2), _vec(conv12_g2), _vec(conv12_be2),
        _vec(conv_am11_w.reshape(-1)),
        mlp_w1.reshape(_C // 4, _C).astype(jnp.bfloat16),
        mlp_b1.reshape(1, -1),
        mlp_w2.reshape(_C, _C // 4).astype(jnp.bfloat16),
        mlp_b2.reshape(1, -1),
        _vec(conv_am21_w.reshape(-1)),
    )
    x1, f, acol, arow = _frontend(X, front_params)

    h1 = _gat_layer(f, acol, arow, gat1_w, gat1_al, gat1_ar,
                    gat1_b.reshape(1, -1), 3, 128, 384, True, False)
    h2 = _gat_layer(h1, acol, arow, gat2_w, gat2_al, gat2_ar,
                    gat2_b.reshape(1, -1), 5, 128, 640, True, False)
    u = _gat_layer(h2, acol, arow, gat3_w, gat3_al, gat3_ar,
                   gat3_b.reshape(1, -1), 3, _C, _C, False, True)

    back_params = (
        _w9(conv22_w1), _vec(conv22_b1), _vec(conv22_g1), _vec(conv22_be1),
        _w9(conv22_w2), _vec(conv22_b2), _vec(conv22_g2), _vec(conv22_be2),
    )
    we = conv_am_end_w.reshape(_C, 2 * _C).astype(jnp.bfloat16)
    out = _backend(u, x1, back_params, we[:, :_C].T, we[:, _C:].T)
    return out.reshape(_B, _HH, _HH, _C).transpose(0, 3, 1, 2)


# speedup vs baseline: 2.0698x; 1.0736x over previous
import functools

import jax
import jax.numpy as jnp
from jax import lax
from jax.experimental import pallas as pl
from jax.experimental.pallas import tpu as pltpu

_C = 64
_HH = 36
_B = 2
_HW = _HH * _HH
_NP = 1408
_T = 1408
_NT = _NP // _T
_F32 = jnp.float32


def _lrelu(v, s):
    return jnp.where(v >= 0, v, s * v)


def _conv3x3(h, w9_ref, b_ref):
    pos = lax.broadcasted_iota(jnp.int32, (1, _HW, 1), 1)
    r = pos // _HH
    c = pos % _HH
    hb = h.astype(jnp.bfloat16)
    acc = None
    for dy in (-1, 0, 1):
        for dx in (-1, 0, 1):
            k = (dy + 1) * 3 + (dx + 1)
            sh = dy * _HH + dx
            shifted = jnp.roll(hb, -sh, axis=1) if sh != 0 else hb
            valid = ((r + dy >= 0) & (r + dy < _HH)
                     & (c + dx >= 0) & (c + dx < _HH))
            masked = jnp.where(valid, shifted, jnp.bfloat16(0))
            term = lax.dot_general(
                masked.reshape(_B * _HW, _C), w9_ref[k],
                (((1,), (0,)), ((), ())), preferred_element_type=_F32)
            acc = term if acc is None else acc + term
    return acc.reshape(_B, _HW, _C) + b_ref[...]


def _bn(h, g_ref, be_ref):
    m = jnp.mean(h, axis=(0, 1), keepdims=True)
    v = jnp.mean((h - m) * (h - m), axis=(0, 1), keepdims=True)
    return (h - m) * lax.rsqrt(v + 1e-5) * g_ref[...] + be_ref[...]


def _dual(h, wa, ba, ga, bea, wb, bb, gb, beb):
    h = jax.nn.relu(_bn(_conv3x3(h, wa, ba), ga, bea))
    h = jax.nn.relu(_bn(_conv3x3(h, wb, bb), gb, beb))
    return h


def _frontend_body(x_ref,
                   cw1, cb1, cg1, cbe1, cw2, cb2, cg2, cbe2,
                   dw1, db1, dg1, dbe1, dw2, db2, dg2, dbe2,
                   am11, mw1, mb1, mw2, mb2, am21,
                   x1_out, f_out, acol_out, arow_out):
    X = x_ref[...]
    x = _dual(X, cw1, cb1, cg1, cbe1, cw2, cb2, cg2, cbe2)
    xb = x.astype(jnp.bfloat16).astype(_F32)

    s1 = jnp.sum(xb * am11[...].astype(jnp.bfloat16).astype(_F32),
                 axis=2, keepdims=True)
    x1 = x * _lrelu(s1, 0.01)
    s = (jnp.max(x1, axis=1, keepdims=True)
         + jnp.mean(x1, axis=1, keepdims=True)).reshape(_B, _C)
    t = jax.nn.relu(lax.dot_general(s.astype(jnp.bfloat16), mw1[...],
                                    (((1,), (1,)), ((), ())),
                                    preferred_element_type=_F32) + mb1[...])
    sg = jax.nn.sigmoid(lax.dot_general(t.astype(jnp.bfloat16), mw2[...],
                                        (((1,), (1,)), ((), ())),
                                        preferred_element_type=_F32) + mb2[...])
    x1 = x1 * sg.reshape(_B, 1, _C)
    x1 = _dual(x1, dw1, db1, dg1, dbe1, dw2, db2, dg2, dbe2)
    x1_out[...] = x1

    s2 = jnp.sum(xb * am21[...].astype(jnp.bfloat16).astype(_F32),
                 axis=2, keepdims=True)
    x2 = x * _lrelu(s2, 0.01)
    zf = jnp.zeros((_B, _NP - _HW, _C), _F32)
    f_out[...] = jnp.concatenate([x2, zf], axis=1)

    acol = jnp.mean(x2, axis=2, keepdims=True)
    za = jnp.zeros((_B, _NP - _HW, 1), _F32)
    acol_out[...] = jnp.concatenate([acol, za], axis=1)
    ones = jnp.full((1, _C), 1.0 / _C, _F32)
    rows = [lax.dot_general(ones, x2[i], (((1,), (1,)), ((), ())),
                            preferred_element_type=_F32,
                             precision=lax.Precision.HIGHEST)
            for i in range(_B)]
    arow = jnp.concatenate(rows, axis=0)
    zr = jnp.zeros((_B, _NP - _HW), _F32)
    arow_out[...] = jnp.concatenate([arow, zr], axis=1).reshape(_B, 1, _NP)


def _frontend(X, p9):
    outs = (
        jax.ShapeDtypeStruct((_B, _HW, _C), _F32),
        jax.ShapeDtypeStruct((_B, _NP, _C), _F32),
        jax.ShapeDtypeStruct((_B, _NP, 1), _F32),
        jax.ShapeDtypeStruct((_B, 1, _NP), _F32),
    )
    return pl.pallas_call(_frontend_body, out_shape=outs)(X, *p9)


def _gat_body(h_ref, acol_ref, arow_ref, w_ref, alb_ref, arb_ref, b_ref,
              out_ref, feat_ref, elr_ref, elc_ref, erc_ref,
              bex_ref, dex_ref, emax_ref,
              *, heads, odim, act, mean_heads):
    j = pl.program_id(1)

    @pl.when(j == 0)
    def _():
        feat = lax.dot_general(h_ref[0], w_ref[...], (((1,), (1,)), ((), ())),
                               preferred_element_type=_F32)
        feat_ref[...] = feat
        elr = lax.dot_general(alb_ref[...], feat, (((1,), (1,)), ((), ())),
                              preferred_element_type=_F32)
        elr_ref[...] = elr
        elc_ref[...] = lax.dot_general(feat, alb_ref[...],
                                       (((1,), (1,)), ((), ())),
                                       preferred_element_type=_F32)
        erc_ref[...] = lax.dot_general(feat, arb_ref[...],
                                       (((1,), (1,)), ((), ())),
                                       preferred_element_type=_F32)
        elmax = jnp.max(elr, axis=1, keepdims=True)
        bex_ref[...] = jnp.exp(elr - elmax)
        dex_ref[...] = jnp.exp(0.2 * (elr - elmax))
        emax_ref[...] = elmax

    d0 = j * _T
    arow = arow_ref[0]
    acol_t = acol_ref[0]
    flag = jnp.max(jnp.abs(arow)) > 0.5
    thr = jnp.where(flag, _F32(0.25), _F32(jnp.inf))
    prod = acol_t * arow
    adj = prod > thr

    acc = None
    for hh in range(heads):
        lo = hh * odim
        feat_h = feat_ref[:, lo:lo + odim]
        feat_t = feat_ref[pl.ds(d0, _T), lo:lo + odim]
        el = elr_ref[hh:hh + 1, :]
        er = erc_ref[pl.ds(d0, _T), hh:hh + 1]
        eld = elc_ref[pl.ds(d0, _T), hh:hh + 1]
        spe = emax_ref[hh:hh + 1, :] + er
        m = _lrelu(spe, 0.2)
        acoef = jnp.exp(spe - m)
        ccoef = jnp.exp(0.2 * spe - m)
        exd = jnp.exp(_lrelu(eld + er, 0.2) - m)
        bex = bex_ref[hh:hh + 1, :]
        dex = dex_ref[hh:hh + 1, :]
        pos = el >= -er
        ex0 = jnp.where(pos, acoef * bex, ccoef * dex)
        ex = jnp.where(adj, ex0, 0.0)
        denom = jnp.sum(ex, axis=1, keepdims=True) + exd
        rst = lax.dot_general(ex, feat_h, (((1,), (0,)), ((), ())),
                              preferred_element_type=_F32)
        rst = (rst + exd * feat_t) / denom + b_ref[:, lo:lo + odim]
        if act:
            rst = jnp.where(rst > 0, rst, jnp.exp(rst) - 1.0)
        if mean_heads:
            acc = rst if acc is None else acc + rst
        else:
            out_ref[0, :, lo:lo + odim] = rst
    if mean_heads:
        out_ref[0, :, :] = acc * (1.0 / heads)


def _blockdiag(a, odim):
    heads = a.shape[0]
    out = jnp.zeros((heads, heads * odim), a.dtype)
    for hh in range(heads):
        out = out.at[hh, hh * odim:(hh + 1) * odim].set(a[hh])
    return out


def _gat_layer(h, acol, arow, W, al, ar, b, heads, odim, width, act,
               mean_heads):
    body = functools.partial(_gat_body, heads=heads, odim=odim, act=act,
                             mean_heads=mean_heads)
    alb = _blockdiag(al, odim)
    arb = _blockdiag(ar, odim)
    return pl.pallas_call(
        body,
        grid=(_B, _NT),
        in_specs=[
            pl.BlockSpec((1, _NP, h.shape[2]), lambda i, j: (i, 0, 0)),
            pl.BlockSpec((1, _T, 1), lambda i, j: (i, j, 0)),
            pl.BlockSpec((1, 1, _NP), lambda i, j: (i, 0, 0)),
            pl.BlockSpec(W.shape, lambda i, j: (0, 0)),
            pl.BlockSpec(alb.shape, lambda i, j: (0, 0)),
            pl.BlockSpec(arb.shape, lambda i, j: (0, 0)),
            pl.BlockSpec(b.shape, lambda i, j: (0, 0)),
        ],
        out_specs=pl.BlockSpec((1, _T, width), lambda i, j: (i, j, 0)),
        out_shape=jax.ShapeDtypeStruct((_B, _NP, width), _F32),
        scratch_shapes=[pltpu.VMEM((_NP, W.shape[0]), _F32),
                        pltpu.VMEM((heads, _NP), _F32),
                        pltpu.VMEM((_NP, heads), _F32),
                        pltpu.VMEM((_NP, heads), _F32),
                        pltpu.VMEM((heads, _NP), _F32),
                        pltpu.VMEM((heads, _NP), _F32),
                        pltpu.VMEM((heads, 1), _F32)],
        compiler_params=pltpu.CompilerParams(
            dimension_semantics=("arbitrary", "arbitrary")),
    )(h, acol, arow, W, alb, arb, b)


def _backend_body(u_ref, x1_ref,
                  ew1, eb1, eg1, ebe1, ew2, eb2, eg2, ebe2,
                  wa_ref, wb_ref, out_ref):
    u = u_ref[...][:, :_HW, :]
    u = _dual(u, ew1, eb1, eg1, ebe1, ew2, eb2, eg2, ebe2)
    x1 = x1_ref[...]
    o = (lax.dot_general(x1.astype(jnp.bfloat16).reshape(_B * _HW, _C),
                         wa_ref[...], (((1,), (0,)), ((), ())),
                         preferred_element_type=_F32)
         + lax.dot_general(u.astype(jnp.bfloat16).reshape(_B * _HW, _C),
                           wb_ref[...], (((1,), (0,)), ((), ())),
                           preferred_element_type=_F32))
    out_ref[...] = _lrelu(o, 0.01).reshape(_B, _HW, _C)


def _backend(u, x1, p9, wa, wb):
    return pl.pallas_call(
        _backend_body,
        out_shape=jax.ShapeDtypeStruct((_B, _HW, _C), _F32),
    )(u, x1, *p9, wa, wb)


def _w9(w):
    return w.transpose(2, 3, 1, 0).reshape(9, _C, _C).astype(jnp.bfloat16)


def _vec(v):
    return v.reshape(1, 1, -1)


def kernel(x, conv_w1, conv_b1, conv_g1, conv_be1, conv_w2, conv_b2, conv_g2,
           conv_be2, conv12_w1, conv12_b1, conv12_g1, conv12_be1, conv12_w2,
           conv12_b2, conv12_g2, conv12_be2, conv22_w1, conv22_b1, conv22_g1,
           conv22_be1, conv22_w2, conv22_b2, conv22_g2, conv22_be2,
           conv_am11_w, mlp_w1, mlp_b1, mlp_w2, mlp_b2, conv_am21_w,
           gat1_w, gat1_al, gat1_ar, gat1_b,
           gat2_w, gat2_al, gat2_ar, gat2_b,
           gat3_w, gat3_al, gat3_ar, gat3_b,
           conv_am_end_w):
    X = x.transpose(0, 2, 3, 1).reshape(_B, _HW, _C)
    front_params = (
        _w9(conv_w1), _vec(conv_b1), _vec(conv_g1), _vec(conv_be1),
        _w9(conv_w2), _vec(conv_b2), _vec(conv_g2), _vec(conv_be2),
        _w9(conv12_w1), _vec(conv12_b1), _vec(conv12_g1), _vec(conv12_be1),
        _w9(conv12_w2), _vec(conv12_b2), _vec(conv12_g2), _vec(conv12_be2),
        _vec(conv_am11_w.reshape(-1)),
        mlp_w1.reshape(_C // 4, _C).astype(jnp.bfloat16),
        mlp_b1.reshape(1, -1),
        mlp_w2.reshape(_C, _C // 4).astype(jnp.bfloat16),
        mlp_b2.reshape(1, -1),
        _vec(conv_am21_w.reshape(-1)),
    )
    x1, f, acol, arow = _frontend(X, front_params)

    h1 = _gat_layer(f, acol, arow, gat1_w, gat1_al, gat1_ar,
                    gat1_b.reshape(1, -1), 3, 128, 384, True, False)
    h2 = _gat_layer(h1, acol, arow, gat2_w, gat2_al, gat2_ar,
                    gat2_b.reshape(1, -1), 5, 128, 640, True, False)
    u = _gat_layer(h2, acol, arow, gat3_w, gat3_al, gat3_ar,
                   gat3_b.reshape(1, -1), 3, _C, _C, False, True)

    back_params = (
        _w9(conv22_w1), _vec(conv22_b1), _vec(conv22_g1), _vec(conv22_be1),
        _w9(conv22_w2), _vec(conv22_b2), _vec(conv22_g2), _vec(conv22_be2),
    )
    we = conv_am_end_w.reshape(_C, 2 * _C).astype(jnp.bfloat16)
    out = _backend(u, x1, back_params, we[:, :_C].T, we[:, _C:].T)
    return out.reshape(_B, _HH, _HH, _C).transpose(0, 3, 1, 2)
